# Initial kernel scaffold; baseline (speedup 1.0000x reference)
import jax
import jax.numpy as jnp
from jax.experimental import pallas as pl


def kernel(head_index, rel_type, tail_index, node_emb, rel_emb):
    raise NotImplementedError("write your pallas kernel here")


# speedup vs baseline: 1.1879x; 1.1879x over previous
import functools

import jax
import jax.numpy as jnp
from jax import lax
from jax.experimental import pallas as pl
from jax.experimental.pallas import tpu as pltpu
from jax.experimental.pallas import tpu_sc as plsc

NC = 2
NS = 16
L = 16
NW = NC * NS

B = 16384
D = 128
BPW = B // NW
CHUNK = 128
NCHUNK = BPW // CHUNK


def _dist_mult_body(head_hbm, rel_t_hbm, tail_hbm, node_hbm, rel_hbm, out_hbm,
                    hidx, ridx, tidx, hbuf, rbuf, tbuf, outv, sem):
    wid = lax.axis_index("s") * NC + lax.axis_index("c")
    base = wid * BPW
    pltpu.sync_copy(head_hbm.at[pl.ds(base, BPW)], hidx)
    pltpu.sync_copy(rel_t_hbm.at[pl.ds(base, BPW)], ridx)
    pltpu.sync_copy(tail_hbm.at[pl.ds(base, BPW)], tidx)
    for c in range(NCHUNK):
        sl = pl.ds(c * CHUNK, CHUNK)
        cp_h = pltpu.async_copy(node_hbm.at[hidx.at[sl]], hbuf, sem)
        cp_r = pltpu.async_copy(rel_hbm.at[ridx.at[sl]], rbuf, sem)
        cp_t = pltpu.async_copy(node_hbm.at[tidx.at[sl]], tbuf, sem)
        cp_h.wait()
        cp_r.wait()
        cp_t.wait()

        lane = lax.iota(jnp.int32, L)

        def group(g, carry, c=c):
            base_e = g * L
            vec = jnp.zeros((L,), jnp.float32)
            for j in range(L):
                e = base_e + j
                acc = hbuf[e, pl.ds(0, L)] * rbuf[e, pl.ds(0, L)] * tbuf[e, pl.ds(0, L)]
                for k in range(1, D // L):
                    ks = pl.ds(k * L, L)
                    acc = acc + hbuf[e, ks] * rbuf[e, ks] * tbuf[e, ks]
                vec = jnp.where(lane == j, jnp.sum(acc), vec)
            outv[pl.ds(c * CHUNK + base_e, L)] = vec
            return carry

        lax.fori_loop(0, CHUNK // L, group, 0)
    pltpu.sync_copy(outv, out_hbm.at[pl.ds(base, BPW)])


@functools.partial(jax.jit, static_argnames=())
def kernel(head_index, rel_type, tail_index, node_emb, rel_emb):
    mesh = plsc.VectorSubcoreMesh(
        core_axis_name="c", subcore_axis_name="s",
        num_cores=NC, num_subcores=NS)
    run = pl.kernel(
        _dist_mult_body,
        out_type=jax.ShapeDtypeStruct((B,), jnp.float32),
        mesh=mesh,
        scratch_types=[
            pltpu.VMEM((BPW,), jnp.int32),
            pltpu.VMEM((BPW,), jnp.int32),
            pltpu.VMEM((BPW,), jnp.int32),
            pltpu.VMEM((CHUNK, D), jnp.float32),
            pltpu.VMEM((CHUNK, D), jnp.float32),
            pltpu.VMEM((CHUNK, D), jnp.float32),
            pltpu.VMEM((BPW,), jnp.float32),
            pltpu.SemaphoreType.DMA,
        ],
        compiler_params=pltpu.CompilerParams(needs_layout_passes=False),
    )
    return run(head_index.astype(jnp.int32), rel_type.astype(jnp.int32),
               tail_index.astype(jnp.int32), node_emb, rel_emb)


# speedup vs baseline: 1.3870x; 1.1676x over previous
import functools

import jax
import jax.numpy as jnp
from jax import lax
from jax.experimental import pallas as pl
from jax.experimental.pallas import tpu as pltpu
from jax.experimental.pallas import tpu_sc as plsc

NC = 2
NS = 16
L = 16
NW = NC * NS

B = 16384
D = 128
BPW = B // NW
CHUNK = 64
NCHUNK = BPW // CHUNK


def _dist_mult_body(head_hbm, rel_t_hbm, tail_hbm, node_hbm, rel_hbm, out_hbm,
                    hidx, ridx, tidx,
                    hbuf0, rbuf0, tbuf0, hbuf1, rbuf1, tbuf1,
                    outv, sem0, sem1, isem):
    wid = lax.axis_index("s") * NC + lax.axis_index("c")
    base = wid * BPW
    ic0 = pltpu.async_copy(head_hbm.at[pl.ds(base, BPW)], hidx, isem)
    ic1 = pltpu.async_copy(rel_t_hbm.at[pl.ds(base, BPW)], ridx, isem)
    ic2 = pltpu.async_copy(tail_hbm.at[pl.ds(base, BPW)], tidx, isem)
    ic0.wait()
    ic1.wait()
    ic2.wait()

    bufs = ((hbuf0, rbuf0, tbuf0), (hbuf1, rbuf1, tbuf1))
    sems = (sem0, sem1)
    lane = lax.iota(jnp.int32, L)

    def fire(c, b):
        sl = pl.ds(c * CHUNK, CHUNK)
        hbuf, rbuf, tbuf = bufs[b]
        pltpu.async_copy(node_hbm.at[hidx.at[sl]], hbuf, sems[b])
        pltpu.async_copy(rel_hbm.at[ridx.at[sl]], rbuf, sems[b])
        pltpu.async_copy(node_hbm.at[tidx.at[sl]], tbuf, sems[b])

    def drain(b):
        sl = pl.ds(0, CHUNK)
        hbuf, rbuf, tbuf = bufs[b]
        pltpu.make_async_copy(node_hbm.at[hidx.at[sl]], hbuf, sems[b]).wait()
        pltpu.make_async_copy(rel_hbm.at[ridx.at[sl]], rbuf, sems[b]).wait()
        pltpu.make_async_copy(node_hbm.at[tidx.at[sl]], tbuf, sems[b]).wait()

    fire(0, 0)
    fire(1, 1)

    def pair(p, carry):
        for b in range(2):
            c = 2 * p + b
            drain(b)
            hbuf, rbuf, tbuf = bufs[b]

            def group(g, carry2, hbuf=hbuf, rbuf=rbuf, tbuf=tbuf, c=c):
                base_e = g * L
                vec = jnp.zeros((L,), jnp.float32)
                for j in range(L):
                    e = base_e + j
                    acc = (hbuf[e, pl.ds(0, L)] * rbuf[e, pl.ds(0, L)]
                           * tbuf[e, pl.ds(0, L)])
                    for k in range(1, D // L):
                        ks = pl.ds(k * L, L)
                        acc = acc + hbuf[e, ks] * rbuf[e, ks] * tbuf[e, ks]
                    vec = jnp.where(lane == j, jnp.sum(acc), vec)
                outv[pl.ds(c * CHUNK + base_e, L)] = vec
                return carry2

            lax.fori_loop(0, CHUNK // L, group, 0)

            @pl.when(p < NCHUNK // 2 - 1)
            def _(c=c, b=b):
                fire(c + 2, b)
        return carry

    lax.fori_loop(0, NCHUNK // 2, pair, 0)
    pltpu.sync_copy(outv, out_hbm.at[pl.ds(base, BPW)])


@functools.partial(jax.jit, static_argnames=())
def kernel(head_index, rel_type, tail_index, node_emb, rel_emb):
    mesh = plsc.VectorSubcoreMesh(
        core_axis_name="c", subcore_axis_name="s",
        num_cores=NC, num_subcores=NS)
    run = pl.kernel(
        _dist_mult_body,
        out_type=jax.ShapeDtypeStruct((B,), jnp.float32),
        mesh=mesh,
        scratch_types=[
            pltpu.VMEM((BPW,), jnp.int32),
            pltpu.VMEM((BPW,), jnp.int32),
            pltpu.VMEM((BPW,), jnp.int32),
            pltpu.VMEM((CHUNK, D), jnp.float32),
            pltpu.VMEM((CHUNK, D), jnp.float32),
            pltpu.VMEM((CHUNK, D), jnp.float32),
            pltpu.VMEM((CHUNK, D), jnp.float32),
            pltpu.VMEM((CHUNK, D), jnp.float32),
            pltpu.VMEM((CHUNK, D), jnp.float32),
            pltpu.VMEM((BPW,), jnp.float32),
            pltpu.SemaphoreType.DMA,
            pltpu.SemaphoreType.DMA,
            pltpu.SemaphoreType.DMA,
        ],
        compiler_params=pltpu.CompilerParams(needs_layout_passes=False),
    )
    return run(head_index.astype(jnp.int32), rel_type.astype(jnp.int32),
               tail_index.astype(jnp.int32), node_emb, rel_emb)
